# baseline (speedup 1.0000x reference)
import functools
import math

import jax
import jax.numpy as jnp
from jax import lax
from jax.experimental import pallas as pl
from jax.experimental.pallas import tpu as pltpu
from jax.experimental.pallas import tpu_sc as plsc

_N_VOCAB = 100000
_PAD = 0
_SMOOTHING = 0.1
_CONF = 1.0 - _SMOOTHING
_S = _SMOOTHING / (_N_VOCAB - 2)
_C_ROW = (_N_VOCAB - 2) * _S * math.log(_S) + _CONF * math.log(_CONF)

_N_TOK = 1024
_ROWS_BLK = 32

_NC = 2
_NS = 16
_NW = _NC * _NS
_LANES = 16
_M_SC = 256
_N_TC = _N_TOK - _M_SC
_RPW = _M_SC // _NW
_HALF = _N_VOCAB // 2


def _tc_kernel(t_smem, t_vmem, p_ref, out_ref):
    i = pl.program_id(0)
    p = p_ref[...]
    rowsum = jnp.sum(p, axis=1, keepdims=True)
    p0 = p[:, 0:1]

    g = jnp.zeros((1, 1), jnp.float32)
    for r in range(_ROWS_BLK):
        t_r = t_smem[r, 0]
        start = pl.multiple_of((t_r // 128) * 128, 128)
        win = p_ref[pl.ds(r, 1), pl.ds(start, 128)]
        lane = lax.broadcasted_iota(jnp.int32, (1, 128), 1)
        val = jnp.sum(jnp.where(lane == (t_r % 128), win, 0.0),
                      axis=(0, 1), keepdims=True)
        g += jnp.where(t_r != _PAD, val, 0.0)

    valid = (t_vmem[...] != _PAD).astype(jnp.float32)
    contrib = valid * (_C_ROW - _S * (rowsum - p0))
    partial = jnp.sum(contrib, axis=(0, 1), keepdims=True)
    partial = partial - (_CONF - _S) * g

    @pl.when(i == 0)
    def _init():
        out_ref[...] = jnp.zeros_like(out_ref)

    out_ref[...] += partial


def _combine_kernel(a_ref, g_ref, out_ref):
    gsum = jnp.sum(g_ref[...], axis=(0, 1), keepdims=True)
    out_ref[...] = a_ref[...] + gsum


_CHUNK = 6144
_NFULL = _N_VOCAB // _CHUNK
_TAIL0 = _NFULL * _CHUNK
_TAILSZ = _N_VOCAB - _TAIL0


def _lane_extract(vec, lane):
    lanes = lax.broadcasted_iota(jnp.int32, (_LANES,), 0)
    return jnp.sum(vec * (lanes == lane).astype(jnp.float32))


def _lane_extract_i32(vec, lane):
    lanes = lax.broadcasted_iota(jnp.int32, (_LANES,), 0)
    return jnp.sum(vec * (lanes == lane).astype(jnp.int32))


def _row_sum(buf, r, n, unroll):
    steps = n // (_LANES * unroll)

    def body(j, acc):
        base = j * (_LANES * unroll)
        for u in range(unroll):
            acc = acc + buf[r, pl.ds(base + u * _LANES, _LANES)]
        return acc

    return lax.fori_loop(0, steps, body, jnp.zeros((_LANES,), jnp.float32))


def _window_extract(buf, r, t_local, size):
    in_range = (t_local >= 0) & (t_local < size)
    tcl = jnp.clip(t_local, 0, size - 1)
    off = tcl // _LANES * _LANES
    val = _lane_extract(buf[r, pl.ds(off, _LANES)], tcl - off)
    return jnp.where(in_range, val, 0.0)


@functools.partial(
    pl.kernel,
    mesh=plsc.VectorSubcoreMesh(core_axis_name="c", subcore_axis_name="s"),
    compiler_params=pltpu.CompilerParams(needs_layout_passes=False),
    out_type=jax.ShapeDtypeStruct((_NW, _LANES), jnp.float32),
    scratch_types=[
        pltpu.VMEM((8, _CHUNK), jnp.float32),
        pltpu.VMEM((8, _TAILSZ), jnp.float32),
        pltpu.VMEM((_LANES,), jnp.int32),
        pltpu.VMEM((_LANES,), jnp.float32),
        pltpu.SemaphoreType.DMA,
    ],
)
def _sc_rows(pred_hbm, t_hbm, out_hbm, buf, tbuf, t_v, ps_v, sem0):
    wid = lax.axis_index("s") * _NC + lax.axis_index("c")
    base_row = _N_TC + wid * _RPW
    group = max(_LANES // _RPW, 1)
    tbase = _N_TC + (wid // group) * group * _RPW
    lane_off = (wid % group) * _RPW
    pltpu.sync_copy(t_hbm.at[pl.ds(tbase, _LANES)], t_v)
    tvals = t_v[...]
    t_rs = [_lane_extract_i32(tvals, lane_off + r) for r in range(8)]

    zero16 = jnp.zeros((_LANES,), jnp.float32)
    zero = jnp.float32(0.0)

    def chunk_body(c, carry):
        accs, vals, p0s = carry
        col = pl.multiple_of(c * _CHUNK, 128)
        pltpu.async_copy(
            pred_hbm.at[pl.ds(base_row, 8), pl.ds(col, _CHUNK)], buf, sem0
        ).wait()
        accs = list(accs)
        vals = list(vals)
        p0s = list(p0s)
        for r in range(8):
            accs[r] = accs[r] + _row_sum(buf, r, _CHUNK, 16)
            vals[r] = vals[r] + _window_extract(buf, r, t_rs[r] - c * _CHUNK,
                                                _CHUNK)
            p00 = _lane_extract(buf[r, pl.ds(0, _LANES)], 0)
            p0s[r] = p0s[r] + jnp.where(c == 0, p00, 0.0)
        return tuple(accs), tuple(vals), tuple(p0s)

    init = ((zero16,) * 8, (zero,) * 8, (zero,) * 8)
    accs, vals, p0s = lax.fori_loop(0, _NFULL, chunk_body, init)

    pltpu.async_copy(
        pred_hbm.at[pl.ds(base_row, 8), pl.ds(_TAIL0, _TAILSZ)], tbuf, sem0
    ).wait()
    ps = zero
    for r in range(8):
        tail_acc = _row_sum(tbuf, r, 1696, 2)
        val = vals[r] + _window_extract(tbuf, r, t_rs[r] - _TAIL0, _TAILSZ)
        rowsum = jnp.sum(accs[r] + tail_acc)
        contrib = _C_ROW - _S * (rowsum - p0s[r]) - (_CONF - _S) * val
        ps = ps + jnp.where(t_rs[r] != _PAD, contrib, 0.0)

    lanes = lax.broadcasted_iota(jnp.int32, (_LANES,), 0)
    ps_v[...] = jnp.where(lanes == 0, jnp.full((_LANES,), ps), 0.0)
    pltpu.sync_copy(ps_v, out_hbm.at[wid])


def kernel(predicts, target):
    n, v = predicts.shape
    t32 = target.astype(jnp.int32)
    t2 = t32.reshape(n, 1)

    a = pl.pallas_call(
        _tc_kernel,
        grid=(_N_TC // _ROWS_BLK,),
        in_specs=[
            pl.BlockSpec((_ROWS_BLK, 1), lambda i: (i, 0),
                         memory_space=pltpu.SMEM),
            pl.BlockSpec((_ROWS_BLK, 1), lambda i: (i, 0)),
            pl.BlockSpec((_ROWS_BLK, v), lambda i: (i, 0)),
        ],
        out_specs=pl.BlockSpec((1, 1), lambda i: (0, 0)),
        out_shape=jax.ShapeDtypeStruct((1, 1), jnp.float32),
    )(t2, t2, predicts)

    g = _sc_rows(predicts, t32)

    loss = pl.pallas_call(
        _combine_kernel,
        in_specs=[
            pl.BlockSpec((1, 1), lambda: (0, 0)),
            pl.BlockSpec((_NW, _LANES), lambda: (0, 0)),
        ],
        out_specs=pl.BlockSpec((1, 1), lambda: (0, 0)),
        out_shape=jax.ShapeDtypeStruct((1, 1), jnp.float32),
    )(a, g)
    return loss[0, 0]
